# baseline (speedup 1.0000x reference)
import functools

import jax
import jax.numpy as jnp
from jax import lax
from jax.experimental import pallas as pl
from jax.experimental.pallas import tpu as pltpu
from jax.experimental.pallas import tpu_sc as plsc

D_MODEL = 64
SCALE = 8.0
NC, NS = 2, 16
NW = NC * NS
C = 128
NBUF = 4
LANES = 16


def kernel(x, table):
    B, L = x.shape
    n = B * L
    per_w = n // NW
    chunks = per_w // C
    groups = chunks // NBUF
    idx = x.T.reshape(n)
    mesh = plsc.VectorSubcoreMesh(core_axis_name="c", subcore_axis_name="s")

    @functools.partial(
        pl.kernel,
        out_type=jax.ShapeDtypeStruct((L * D_MODEL, B), jnp.float32),
        mesh=mesh,
        compiler_params=pltpu.CompilerParams(
            use_tc_tiling_on_sc=False, needs_layout_passes=False
        ),
        scratch_types=[
            pltpu.VMEM((per_w,), jnp.int32),
            pltpu.VMEM((NBUF, C, D_MODEL), jnp.float32),
            pltpu.VMEM((NBUF, D_MODEL, C), jnp.float32),
            pltpu.SemaphoreType.DMA((NBUF,)),
            pltpu.SemaphoreType.DMA((NBUF,)),
            pltpu.SemaphoreType.DMA,
        ],
    )
    def gather_scale(table_hbm, idx_hbm, out_hbm, idx_v, buf, xbuf, gsem, ssem, isem):
        wid = lax.axis_index("s") * NC + lax.axis_index("c")
        base = pl.multiple_of(wid * per_w, per_w)
        pltpu.async_copy(idx_hbm.at[pl.ds(base, per_w)], idx_v, isem).wait()

        row0 = lax.iota(jnp.int32, LANES)
        rows_t = [row0 + t * LANES for t in range(C // LANES)]

        def out_block(j):
            gp0 = base + j * C
            l = gp0 // B
            i0 = pl.multiple_of(gp0 % B, C)
            return out_hbm.at[pl.ds(l * D_MODEL, D_MODEL), pl.ds(i0, C)]

        @pl.loop(0, groups)
        def _(g):
            j0 = g * NBUF
            fired = []
            for b in range(NBUF):
                off = pl.multiple_of((j0 + b) * C, C)
                fired.append(
                    pltpu.async_copy(
                        table_hbm.at[idx_v.at[pl.ds(off, C)]],
                        buf.at[b],
                        gsem.at[b],
                    )
                )
            for b in range(NBUF):
                j = j0 + b

                @pl.when(g > 0)
                def _():
                    pltpu.make_async_copy(
                        xbuf.at[b], out_block(j), ssem.at[b]
                    ).wait()

                fired[b].wait()

                @pl.loop(0, D_MODEL)
                def _(d):
                    cols = jnp.full((LANES,), d, jnp.int32)
                    for t in range(C // LANES):
                        vals = plsc.load_gather(buf.at[b], [rows_t[t], cols])
                        xbuf[b, d, pl.ds(t * LANES, LANES)] = vals * SCALE

                pltpu.async_copy(xbuf.at[b], out_block(j), ssem.at[b])

        for b in range(NBUF):
            pltpu.make_async_copy(
                xbuf.at[b], out_block((groups - 1) * NBUF + b), ssem.at[b]
            ).wait()

    out = gather_scale(table, idx)
    return out.reshape(L, D_MODEL, B).transpose(2, 0, 1)


# speedup vs baseline: 1.7509x; 1.7509x over previous
import functools

import jax
import jax.numpy as jnp
from jax import lax
from jax.experimental import pallas as pl
from jax.experimental.pallas import tpu as pltpu
from jax.experimental.pallas import tpu_sc as plsc

D_MODEL = 64
SCALE = 8.0
NC, NS = 2, 16
NW = NC * NS
C = 128
NBUF = 8


def kernel(x, table):
    B, L = x.shape
    n = B * L
    V = table.shape[0]
    per_w = n // NW
    chunks = per_w // C
    groups = chunks // NBUF
    idx = x.reshape(n)
    table_lin = lax.optimization_barrier(table.reshape(V // 2, 2 * D_MODEL))
    table_lin = table_lin.reshape(V, D_MODEL)
    mesh = plsc.VectorSubcoreMesh(core_axis_name="c", subcore_axis_name="s")

    @functools.partial(
        pl.kernel,
        out_type=jax.ShapeDtypeStruct((n, D_MODEL), jnp.float32),
        mesh=mesh,
        compiler_params=pltpu.CompilerParams(use_tc_tiling_on_sc=False),
        scratch_types=[
            pltpu.VMEM((per_w,), jnp.int32),
            pltpu.VMEM((NBUF, C, D_MODEL), jnp.float32),
            pltpu.SemaphoreType.DMA((NBUF,)),
            pltpu.SemaphoreType.DMA((NBUF,)),
            pltpu.SemaphoreType.DMA,
        ],
    )
    def gather_scale(table_hbm, idx_hbm, out_hbm, idx_v, rows_v, gsem, ssem, isem):
        wid = lax.axis_index("s") * NC + lax.axis_index("c")
        base = pl.multiple_of(wid * per_w, per_w)
        pltpu.async_copy(idx_hbm.at[pl.ds(base, per_w)], idx_v, isem).wait()

        @pl.loop(0, groups)
        def _(g):
            j0 = g * NBUF
            fired = []
            for b in range(NBUF):
                off = pl.multiple_of((j0 + b) * C, C)

                @pl.when(g > 0)
                def _():
                    pltpu.make_async_copy(
                        rows_v.at[b],
                        out_hbm.at[pl.ds(base + off - NBUF * C, C)],
                        ssem.at[b],
                    ).wait()

                fired.append(
                    pltpu.async_copy(
                        table_hbm.at[idx_v.at[pl.ds(off, C)]],
                        rows_v.at[b],
                        gsem.at[b],
                    )
                )
            for b in range(NBUF):
                off = pl.multiple_of((j0 + b) * C, C)
                fired[b].wait()

                @pl.loop(0, C)
                def _(r):
                    for c0 in range(0, D_MODEL, 16):
                        rows_v[b, r, pl.ds(c0, 16)] = (
                            rows_v[b, r, pl.ds(c0, 16)] * SCALE
                        )

                pltpu.async_copy(
                    rows_v.at[b],
                    out_hbm.at[pl.ds(base + off, C)],
                    ssem.at[b],
                )

        for b in range(NBUF):
            off = ((groups - 1) * NBUF + b) * C
            pltpu.make_async_copy(
                rows_v.at[b],
                out_hbm.at[pl.ds(base + off, C)],
                ssem.at[b],
            ).wait()

    out = gather_scale(table_lin, idx)
    return out.reshape(B, L, D_MODEL)


# speedup vs baseline: 2.2683x; 1.2955x over previous
import functools

import jax
import jax.numpy as jnp
from jax import lax
from jax.experimental import pallas as pl
from jax.experimental.pallas import tpu as pltpu
from jax.experimental.pallas import tpu_sc as plsc

D_MODEL = 64
SCALE = 8.0
NC, NS = 2, 16
NW = NC * NS
C = 128
NBUF = 8


def kernel(x, table):
    B, L = x.shape
    n = B * L
    V = table.shape[0]
    per_w = n // NW
    chunks = per_w // C
    groups = chunks // NBUF
    idx = x.reshape(n) * 2
    table_lin = lax.optimization_barrier(jnp.pad(table, ((0, 0), (0, D_MODEL))))
    table_lin = table_lin.reshape(2 * V, D_MODEL)
    mesh = plsc.VectorSubcoreMesh(core_axis_name="c", subcore_axis_name="s")

    @functools.partial(
        pl.kernel,
        out_type=jax.ShapeDtypeStruct((n, 2 * D_MODEL), jnp.float32),
        mesh=mesh,
        compiler_params=pltpu.CompilerParams(use_tc_tiling_on_sc=False),
        scratch_types=[
            pltpu.VMEM((per_w,), jnp.int32),
            pltpu.VMEM((NBUF, C, D_MODEL), jnp.float32),
            pltpu.SemaphoreType.DMA((NBUF,)),
            pltpu.SemaphoreType.DMA((NBUF,)),
            pltpu.SemaphoreType.DMA,
        ],
    )
    def gather_scale(table_hbm, idx_hbm, out_hbm, idx_v, rows_v, gsem, ssem, isem):
        wid = lax.axis_index("s") * NC + lax.axis_index("c")
        base = pl.multiple_of(wid * per_w, per_w)
        pltpu.async_copy(idx_hbm.at[pl.ds(base, per_w)], idx_v, isem).wait()

        @pl.loop(0, groups)
        def _(g):
            j0 = g * NBUF
            fired = []
            for b in range(NBUF):
                off = pl.multiple_of((j0 + b) * C, C)

                @pl.when(g > 0)
                def _():
                    pltpu.make_async_copy(
                        rows_v.at[b],
                        out_hbm.at[pl.ds(base + off - NBUF * C, C), pl.ds(0, D_MODEL)],
                        ssem.at[b],
                    ).wait()

                fired.append(
                    pltpu.async_copy(
                        table_hbm.at[idx_v.at[pl.ds(off, C)]],
                        rows_v.at[b],
                        gsem.at[b],
                    )
                )
            for b in range(NBUF):
                off = pl.multiple_of((j0 + b) * C, C)
                fired[b].wait()

                @pl.loop(0, C)
                def _(r):
                    for c0 in range(0, D_MODEL, 16):
                        rows_v[b, r, pl.ds(c0, 16)] = (
                            rows_v[b, r, pl.ds(c0, 16)] * SCALE
                        )

                pltpu.async_copy(
                    rows_v.at[b],
                    out_hbm.at[pl.ds(base + off, C), pl.ds(0, D_MODEL)],
                    ssem.at[b],
                )

        for b in range(NBUF):
            off = ((groups - 1) * NBUF + b) * C
            pltpu.make_async_copy(
                rows_v.at[b],
                out_hbm.at[pl.ds(base + off, C), pl.ds(0, D_MODEL)],
                ssem.at[b],
            ).wait()

    out = gather_scale(table_lin, idx)
    return out[:, :D_MODEL].reshape(B, L, D_MODEL)
